# Initial kernel scaffold; baseline (speedup 1.0000x reference)
import jax
import jax.numpy as jnp
from jax.experimental import pallas as pl


def kernel(x, edge_index, batch, W0, b0, ln_g, ln_b, t, W1, b1, mg, mb, W2, b2):
    raise NotImplementedError("write your pallas kernel here")


# speedup vs baseline: 8.4907x; 8.4907x over previous
import functools

import jax
import jax.numpy as jnp
from jax import lax
from jax.experimental import pallas as pl
from jax.experimental.pallas import tpu as pltpu
from jax.experimental.pallas import tpu_sc as plsc

N_NODES = 10000
N_EDGES = 320000
F = 128
N_LAYERS = 6
N_GRAPHS = 16
EPS = 1e-7

NC = 2
NS = 16
CW = 128
EDGES_PT = -(-N_EDGES // (NS * CW)) * CW
NCHUNK = EDGES_PT // CW
E_PAD = EDGES_PT * NS
ROWS_PAD = 10240
SLAB = ROWS_PAD // NS

BR = 2000
NB = N_NODES // BR
HF = F // 2


def _ln(z, g, b, eps=1e-5):
    m = jnp.mean(z, axis=-1, keepdims=True)
    v = jnp.mean((z - m) ** 2, axis=-1, keepdims=True)
    return (z - m) * lax.rsqrt(v + eps) * g + b


def _enc_body(x_ref, w_ref, b_ref, o_ref):
    o_ref[...] = (
        jnp.dot(x_ref[...], w_ref[...], preferred_element_type=jnp.float32)
        + b_ref[...]
    )


def _encode(x, W0, b0):
    return pl.pallas_call(
        _enc_body,
        grid=(NB,),
        in_specs=[
            pl.BlockSpec((BR, F), lambda i: (i, 0)),
            pl.BlockSpec((F, F), lambda i: (0, 0)),
            pl.BlockSpec((1, F), lambda i: (0, 0)),
        ],
        out_specs=pl.BlockSpec((BR, F), lambda i: (i, 0)),
        out_shape=jax.ShapeDtypeStruct((N_NODES, F), jnp.float32),
    )(x, W0, b0.reshape(1, F))


def _pre_body(h_ref, g_ref, b_ref, t_ref, hn_ref, p_ref, e_ref):
    hn = jnp.maximum(_ln(h_ref[...], g_ref[...], b_ref[...]), 0.0)
    hn_ref[...] = hn
    m = hn + EPS
    e = jnp.exp(m * t_ref[0, 0])
    e_ref[...] = e
    p_ref[...] = m * e


def _pre(h, g, b, t):
    return pl.pallas_call(
        _pre_body,
        grid=(NB,),
        in_specs=[
            pl.BlockSpec((BR, F), lambda i: (i, 0)),
            pl.BlockSpec((1, F), lambda i: (0, 0)),
            pl.BlockSpec((1, F), lambda i: (0, 0)),
            pl.BlockSpec((1, 1), lambda i: (0, 0)),
        ],
        out_specs=[pl.BlockSpec((BR, F), lambda i: (i, 0))] * 3,
        out_shape=[jax.ShapeDtypeStruct((N_NODES, F), jnp.float32)] * 3,
    )(h, g.reshape(1, F), b.reshape(1, F), t.reshape(1, 1))


def _post_body(num0_ref, num1_ref, den0_ref, den1_ref, hn_ref, h_ref,
               w1_ref, b1_ref, mg_ref, mb_ref, w2_ref, b2_ref, o_ref):
    num = jnp.concatenate([num0_ref[...], num1_ref[...]], axis=1)
    den = jnp.concatenate([den0_ref[...], den1_ref[...]], axis=1)
    aggr = jnp.where(den > 0.0, num / den, 0.0)
    out = aggr + hn_ref[...]
    z = jnp.dot(out, w1_ref[...], preferred_element_type=jnp.float32) + b1_ref[...]
    z = jnp.maximum(_ln(z, mg_ref[...], mb_ref[...]), 0.0)
    o_ref[...] = (
        jnp.dot(z, w2_ref[...], preferred_element_type=jnp.float32)
        + b2_ref[...]
        + h_ref[...]
    )


def _post(num0, num1, den0, den1, hn, h, W1, b1, mg, mb, W2, b2):
    return pl.pallas_call(
        _post_body,
        grid=(NB,),
        in_specs=[
            pl.BlockSpec((BR, HF), lambda i: (i, 0)),
            pl.BlockSpec((BR, HF), lambda i: (i, 0)),
            pl.BlockSpec((BR, HF), lambda i: (i, 0)),
            pl.BlockSpec((BR, HF), lambda i: (i, 0)),
            pl.BlockSpec((BR, F), lambda i: (i, 0)),
            pl.BlockSpec((BR, F), lambda i: (i, 0)),
            pl.BlockSpec((F, 2 * F), lambda i: (0, 0)),
            pl.BlockSpec((1, 2 * F), lambda i: (0, 0)),
            pl.BlockSpec((1, 2 * F), lambda i: (0, 0)),
            pl.BlockSpec((1, 2 * F), lambda i: (0, 0)),
            pl.BlockSpec((2 * F, F), lambda i: (0, 0)),
            pl.BlockSpec((1, F), lambda i: (0, 0)),
        ],
        out_specs=pl.BlockSpec((BR, F), lambda i: (i, 0)),
        out_shape=jax.ShapeDtypeStruct((N_NODES, F), jnp.float32),
    )(num0, num1, den0, den1, hn, h, W1, b1.reshape(1, 2 * F),
      mg.reshape(1, 2 * F), mb.reshape(1, 2 * F), W2, b2.reshape(1, F))


def _pool_body(h_ref, bat_ref, o_ref):
    @pl.when(pl.program_id(0) == 0)
    def _():
        o_ref[...] = jnp.zeros_like(o_ref)

    onehot = (bat_ref[...] ==
              lax.broadcasted_iota(jnp.int32, (1, N_GRAPHS), 1)).astype(jnp.float32)
    o_ref[...] += lax.dot_general(
        onehot, h_ref[...], (((0,), (0,)), ((), ())),
        preferred_element_type=jnp.float32)


def _pool(h, batch):
    return pl.pallas_call(
        _pool_body,
        grid=(NB,),
        in_specs=[
            pl.BlockSpec((BR, F), lambda i: (i, 0)),
            pl.BlockSpec((BR, 1), lambda i: (i, 0)),
        ],
        out_specs=pl.BlockSpec((N_GRAPHS, F), lambda i: (0, 0)),
        out_shape=jax.ShapeDtypeStruct((N_GRAPHS, F), jnp.float32),
    )(h, batch.reshape(N_NODES, 1))


def _sc_body(p_hbm, e_hbm, srcA_hbm, srcB_hbm, dst_hbm,
             num0_out, num1_out, den0_out, den1_out,
             srcA_v, srcB_v, dst_v, rows_v, zbuf, acc, gsem):
    c = lax.axis_index("c")
    s = lax.axis_index("s")

    pltpu.sync_copy(srcA_hbm.at[s], srcA_v)
    pltpu.sync_copy(srcB_hbm.at[s], srcB_v)
    pltpu.sync_copy(dst_hbm.at[s], dst_v)

    for r in range(16):
        for q in range(HF // 16):
            zbuf[r, pl.ds(q * 16, 16)] = jnp.zeros((16,), jnp.float32)

    def stage(tab, src_v, out_ref):
        @pl.loop(0, SLAB // 16)
        def _(j):
            pltpu.sync_copy(zbuf, acc.at[pl.ds(s * SLAB + j * 16, 16)])

        plsc.subcore_barrier()

        @pl.loop(0, NCHUNK)
        def _(j):
            pltpu.async_copy(tab.at[src_v.at[j]], rows_v, gsem).wait()
            pltpu.sync_copy(rows_v, acc.at[dst_v.at[j]], add=True)

        plsc.subcore_barrier()

        pltpu.sync_copy(acc.at[pl.ds(s * SLAB, SLAB)],
                        out_ref.at[pl.ds(s * SLAB, SLAB)])

    @pl.when(c == 0)
    def _():
        stage(p_hbm, srcA_v, num0_out)
        stage(p_hbm, srcB_v, num1_out)

    @pl.when(c == 1)
    def _():
        stage(e_hbm, srcA_v, den0_out)
        stage(e_hbm, srcB_v, den1_out)


@functools.cache
def _build_sc_aggregate():
    return pl.kernel(
        _sc_body,
        out_type=[jax.ShapeDtypeStruct((ROWS_PAD, HF), jnp.float32)] * 4,
        mesh=plsc.VectorSubcoreMesh(core_axis_name="c", subcore_axis_name="s",
                                    num_cores=NC, num_subcores=NS),
        scratch_types=[
            pltpu.VMEM((NCHUNK, CW), jnp.int32),
            pltpu.VMEM((NCHUNK, CW), jnp.int32),
            pltpu.VMEM((NCHUNK, CW), jnp.int32),
            pltpu.VMEM((CW, HF), jnp.float32),
            pltpu.VMEM((16, HF), jnp.float32),
            pltpu.VMEM_SHARED((ROWS_PAD, HF), jnp.float32),
            pltpu.SemaphoreType.DMA,
        ],
        compiler_params=pltpu.CompilerParams(use_tc_tiling_on_sc=False),
    )


def _sc_aggregate(p, e, srcA, srcB, dstI):
    return _build_sc_aggregate()(p.reshape(2 * N_NODES, HF),
                                 e.reshape(2 * N_NODES, HF),
                                 srcA, srcB, dstI)


def kernel(x, edge_index, batch, W0, b0, ln_g, ln_b, t, W1, b1, mg, mb,
           W2, b2):
    src = edge_index[0].astype(jnp.int32)
    dst = edge_index[1].astype(jnp.int32)
    src2 = jnp.pad(2 * src, (0, E_PAD - N_EDGES))
    srcA = src2.reshape(NS, NCHUNK, CW)
    srcB = (src2 + 1).reshape(NS, NCHUNK, CW)
    dstI = jnp.pad(dst, (0, E_PAD - N_EDGES),
                   constant_values=N_NODES).reshape(NS, NCHUNK, CW)

    h = _encode(x, W0, b0)
    for i in range(N_LAYERS):
        hn, p, e = _pre(h, ln_g[i], ln_b[i], t[i])
        num0, num1, den0, den1 = _sc_aggregate(p, e, srcA, srcB, dstI)
        h = _post(num0, num1, den0, den1, hn, h,
                  W1[i], b1[i], mg[i], mb[i], W2[i], b2[i])
    return _pool(h, batch.astype(jnp.int32))
